# baseline (speedup 1.0000x reference)
import functools

import jax
import jax.numpy as jnp
from jax.experimental import pallas as pl
from jax.experimental.pallas import tpu as pltpu


def _chamfer_kernel(p1_ref, p2_ref, out_ref, row_acc, col_acc, *,
                    n1, n2, npad1, npad2, ti, tj, ni, nj):
    i = pl.program_id(0)
    j = pl.program_id(1)

    p1 = p1_ref[...]
    p2 = p2_ref[...]

    acc = jnp.zeros((ti, tj), jnp.float32)
    for d in range(3):
        diff = p1[:, d][:, None] - p2[d, :][None, :]
        acc = acc + diff * diff

    row_m = jnp.min(acc, axis=1)[:, None]
    col_m = jnp.min(acc, axis=0)[None, :]

    @pl.when(j == 0)
    def _():
        row_acc[pl.ds(i * ti, ti), :] = row_m

    @pl.when(j > 0)
    def _():
        row_acc[pl.ds(i * ti, ti), :] = jnp.minimum(
            row_acc[pl.ds(i * ti, ti), :], row_m)

    @pl.when(i == 0)
    def _():
        col_acc[:, pl.ds(j * tj, tj)] = col_m

    @pl.when(i > 0)
    def _():
        col_acc[:, pl.ds(j * tj, tj)] = jnp.minimum(
            col_acc[:, pl.ds(j * tj, tj)], col_m)

    @pl.when((i == ni - 1) & (j == nj - 1))
    def _():
        rm = row_acc[...]
        rvalid = jax.lax.broadcasted_iota(jnp.int32, (npad1, 1), 0) < n1
        s1 = jnp.sum(jnp.where(rvalid, jnp.sqrt(rm), 0.0))
        cm = col_acc[...]
        cvalid = jax.lax.broadcasted_iota(jnp.int32, (1, npad2), 1) < n2
        s2 = jnp.sum(jnp.where(cvalid, jnp.sqrt(cm), 0.0))
        out_ref[...] = (s1 + s2)[None, None]


def kernel(points1, points2):
    n1 = points1.shape[0]
    n2 = points2.shape[0]
    ti = 2048
    tj = 5120
    npad1 = ((n1 + ti - 1) // ti) * ti
    npad2 = ((n2 + tj - 1) // tj) * tj
    ni = npad1 // ti
    nj = npad2 // tj

    p1p = jnp.zeros((npad1, 8), jnp.float32)
    p1p = p1p.at[:n1, :3].set(points1.astype(jnp.float32))
    p1p = p1p.at[n1:, :3].set(jnp.inf)

    p2p = jnp.zeros((8, npad2), jnp.float32)
    p2p = p2p.at[:3, :n2].set(points2.astype(jnp.float32).T)
    p2p = p2p.at[:3, n2:].set(jnp.inf)

    body = functools.partial(
        _chamfer_kernel,
        n1=n1, n2=n2, npad1=npad1, npad2=npad2,
        ti=ti, tj=tj, ni=ni, nj=nj)

    out = pl.pallas_call(
        body,
        grid=(ni, nj),
        in_specs=[
            pl.BlockSpec((ti, 8), lambda i, j: (i, 0)),
            pl.BlockSpec((8, tj), lambda i, j: (0, j)),
        ],
        out_specs=pl.BlockSpec((1, 1), lambda i, j: (0, 0)),
        out_shape=jax.ShapeDtypeStruct((1, 1), jnp.float32),
        scratch_shapes=[
            pltpu.VMEM((npad1, 1), jnp.float32),
            pltpu.VMEM((1, npad2), jnp.float32),
        ],
        compiler_params=pltpu.CompilerParams(
            dimension_semantics=("arbitrary", "arbitrary"),
        ),
    )(p1p, p2p)
    return out[0, 0]


# speedup vs baseline: 1.2171x; 1.2171x over previous
import functools

import jax
import jax.numpy as jnp
from jax.experimental import pallas as pl
from jax.experimental.pallas import tpu as pltpu


def _chamfer_kernel(p1_ref, p2_ref, out_ref, row_acc, col_acc, *,
                    n1, n2, npad1, npad2, ti, tj, ni, nj):
    i = pl.program_id(0)
    j = pl.program_id(1)

    p1 = p1_ref[...]
    p2 = p2_ref[...]

    acc = jnp.zeros((ti, tj), jnp.float32)
    for d in range(3):
        diff = p1[:, d][:, None] - p2[d, :][None, :]
        acc = acc + diff * diff

    row_m = jnp.min(acc, axis=1)[:, None]
    col_m = jnp.min(acc, axis=0)[None, :]

    @pl.when(j == 0)
    def _():
        row_acc[pl.ds(i * ti, ti), :] = row_m

    @pl.when(j > 0)
    def _():
        row_acc[pl.ds(i * ti, ti), :] = jnp.minimum(
            row_acc[pl.ds(i * ti, ti), :], row_m)

    @pl.when(i == 0)
    def _():
        col_acc[:, pl.ds(j * tj, tj)] = col_m

    @pl.when(i > 0)
    def _():
        col_acc[:, pl.ds(j * tj, tj)] = jnp.minimum(
            col_acc[:, pl.ds(j * tj, tj)], col_m)

    @pl.when((i == ni - 1) & (j == nj - 1))
    def _():
        rm = row_acc[...]
        rvalid = jax.lax.broadcasted_iota(jnp.int32, (npad1, 1), 0) < n1
        s1 = jnp.sum(jnp.where(rvalid, jnp.sqrt(rm), 0.0))
        cm = col_acc[...]
        cvalid = jax.lax.broadcasted_iota(jnp.int32, (1, npad2), 1) < n2
        s2 = jnp.sum(jnp.where(cvalid, jnp.sqrt(cm), 0.0))
        out_ref[...] = (s1 + s2)[None, None]


def kernel(points1, points2):
    n1 = points1.shape[0]
    n2 = points2.shape[0]
    ti = 2560
    tj = 2048
    npad1 = ((n1 + ti - 1) // ti) * ti
    npad2 = ((n2 + tj - 1) // tj) * tj
    ni = npad1 // ti
    nj = npad2 // tj

    p1p = jnp.zeros((npad1, 8), jnp.float32)
    p1p = p1p.at[:n1, :3].set(points1.astype(jnp.float32))
    p1p = p1p.at[n1:, :3].set(jnp.inf)

    p2p = jnp.zeros((8, npad2), jnp.float32)
    p2p = p2p.at[:3, :n2].set(points2.astype(jnp.float32).T)
    p2p = p2p.at[:3, n2:].set(jnp.inf)

    body = functools.partial(
        _chamfer_kernel,
        n1=n1, n2=n2, npad1=npad1, npad2=npad2,
        ti=ti, tj=tj, ni=ni, nj=nj)

    out = pl.pallas_call(
        body,
        grid=(ni, nj),
        in_specs=[
            pl.BlockSpec((ti, 8), lambda i, j: (i, 0)),
            pl.BlockSpec((8, tj), lambda i, j: (0, j)),
        ],
        out_specs=pl.BlockSpec((1, 1), lambda i, j: (0, 0)),
        out_shape=jax.ShapeDtypeStruct((1, 1), jnp.float32),
        scratch_shapes=[
            pltpu.VMEM((npad1, 1), jnp.float32),
            pltpu.VMEM((1, npad2), jnp.float32),
        ],
        compiler_params=pltpu.CompilerParams(
            dimension_semantics=("arbitrary", "arbitrary"),
        ),
    )(p1p, p2p)
    return out[0, 0]
